# baseline (speedup 1.0000x reference)
import functools

import jax
import jax.numpy as jnp
from jax import lax
from jax.experimental import pallas as pl
from jax.experimental.pallas import tpu as pltpu
from jax.experimental.pallas import tpu_sc as plsc

jax.config.update("jax_enable_x64", True)

_L = 16

_MESH = plsc.VectorSubcoreMesh(core_axis_name="c", subcore_axis_name="s")


def _sc_body(src_hbm, ten_hbm, idx_hbm, out_hbm, src_v, ten_v, idx_v, out_v):
    c = lax.axis_index("c")
    s = lax.axis_index("s")

    @pl.when(jnp.logical_and(c == 0, s == 0))
    def _():
        pltpu.sync_copy(src_hbm, src_v)
        pltpu.sync_copy(ten_hbm, ten_v)
        pltpu.sync_copy(idx_hbm, idx_v)
        lanes = lax.iota(jnp.int32, _L)
        hit = lanes == idx_v[...]
        out_v[...] = ten_v[...] * 0.5 + jnp.where(hit, src_v[...], 0.0)
        pltpu.sync_copy(out_v, out_hbm)


@jax.jit
def _scatter_add_halve(src16, ten16, idx16):
    run = pl.kernel(
        _sc_body,
        out_type=jax.ShapeDtypeStruct((_L,), jnp.float32),
        mesh=_MESH,
        scratch_types=[
            pltpu.VMEM((_L,), jnp.float32),
            pltpu.VMEM((_L,), jnp.float32),
            pltpu.VMEM((_L,), jnp.int32),
            pltpu.VMEM((_L,), jnp.float32),
        ],
    )
    return run(src16, ten16, idx16)


def kernel(source, tensor, index):
    src16 = jnp.broadcast_to(source.astype(jnp.float32), (_L,))
    ten16 = jnp.pad(tensor.astype(jnp.float32), (0, _L - tensor.shape[0]))
    idx16 = jnp.broadcast_to(index.astype(jnp.int32), (_L,))
    out16 = _scatter_add_halve(src16, ten16, idx16)
    out = out16[:1].astype(jnp.float64)
    return (source, out)
